# baseline (speedup 1.0000x reference)
import functools

import jax
import jax.numpy as jnp
from jax import lax
from jax.experimental import pallas as pl
from jax.experimental.pallas import tpu as pltpu
from jax.experimental.pallas import tpu_sc as plsc

BATCH = 16384
LIST_LEN = 50
POSITIONS = 50
TABLE_PAD = 64

N = BATCH * LIST_LEN
NC, NS, L = 2, 16, 16
NW = NC * NS
PER_W = N // NW
UNROLL = 8
VEC_ITERS = PER_W // (L * UNROLL)


def _build():
    mesh = plsc.VectorSubcoreMesh(core_axis_name="c", subcore_axis_name="s")

    @functools.partial(
        pl.kernel,
        mesh=mesh,
        out_type=jax.ShapeDtypeStruct((N,), jnp.float32),
        compiler_params=pltpu.CompilerParams(needs_layout_passes=False),
        scratch_types=[
            pltpu.VMEM((TABLE_PAD,), jnp.float32),
            pltpu.VMEM((PER_W,), jnp.int32),
            pltpu.VMEM((PER_W,), jnp.float32),
        ],
    )
    def gather_kernel(pos_hbm, model_hbm, out_hbm, table_v, idx_v, vals_v):
        wid = lax.axis_index("s") * NC + lax.axis_index("c")
        base = wid * PER_W
        pltpu.sync_copy(model_hbm, table_v)
        pltpu.sync_copy(pos_hbm.at[pl.ds(base, PER_W)], idx_v)

        def body(i, carry):
            for u in range(UNROLL):
                off = (i * UNROLL + u) * L
                idx = idx_v[pl.ds(off, L)]
                vals_v[pl.ds(off, L)] = plsc.load_gather(table_v, [idx])
            return carry

        lax.fori_loop(0, VEC_ITERS, body, 0)
        pltpu.sync_copy(vals_v, out_hbm.at[pl.ds(base, PER_W)])

    return gather_kernel


_GATHER = _build()


@jax.jit
def kernel(position, model):
    pos_flat = position.reshape(N)
    model_pad = jnp.zeros((TABLE_PAD,), jnp.float32).at[:POSITIONS].set(model)
    out = _GATHER(pos_flat, model_pad)
    return out.reshape(BATCH, LIST_LEN)


# speedup vs baseline: 1.0257x; 1.0257x over previous
import functools

import jax
import jax.numpy as jnp
from jax import lax
from jax.experimental import pallas as pl
from jax.experimental.pallas import tpu as pltpu
from jax.experimental.pallas import tpu_sc as plsc

BATCH = 16384
LIST_LEN = 50
POSITIONS = 50
TABLE_PAD = 64

N = BATCH * LIST_LEN
NC, NS, L = 2, 16, 16
NW = NC * NS
PER_W = N // NW
UNROLL = 8
VEC_ITERS = PER_W // (L * UNROLL)


def _build():
    mesh = plsc.VectorSubcoreMesh(core_axis_name="c", subcore_axis_name="s")

    @functools.partial(
        pl.kernel,
        mesh=mesh,
        out_type=jax.ShapeDtypeStruct((N,), jnp.float32),
        compiler_params=pltpu.CompilerParams(needs_layout_passes=False),
        scratch_types=[
            pltpu.VMEM((TABLE_PAD,), jnp.float32),
            pltpu.VMEM((PER_W,), jnp.int32),
            pltpu.VMEM((PER_W,), jnp.float32),
        ],
    )
    def gather_kernel(pos_hbm, model_hbm, out_hbm, table_v, idx_v, vals_v):
        wid = lax.axis_index("s") * NC + lax.axis_index("c")
        base = wid * PER_W
        pltpu.sync_copy(model_hbm, table_v.at[pl.ds(0, POSITIONS)])
        pltpu.sync_copy(pos_hbm.at[pl.ds(base, PER_W)], idx_v)

        def body(i, carry):
            for u in range(UNROLL):
                off = (i * UNROLL + u) * L
                idx = idx_v[pl.ds(off, L)]
                vals_v[pl.ds(off, L)] = plsc.load_gather(table_v, [idx])
            return carry

        lax.fori_loop(0, VEC_ITERS, body, 0)
        pltpu.sync_copy(vals_v, out_hbm.at[pl.ds(base, PER_W)])

    return gather_kernel


_GATHER = _build()


@jax.jit
def kernel(position, model):
    out = _GATHER(position.reshape(N), model)
    return out.reshape(BATCH, LIST_LEN)


# speedup vs baseline: 1.2893x; 1.2570x over previous
import functools

import jax
import jax.numpy as jnp
from jax import lax
from jax.experimental import pallas as pl
from jax.experimental.pallas import tpu as pltpu
from jax.experimental.pallas import tpu_sc as plsc

BATCH = 16384
LIST_LEN = 50
POSITIONS = 50
TABLE_PAD = 64

NC, NS, L = 2, 16, 16
NW = NC * NS
ROWS_W = BATCH // NW
CHUNK = 256
NCHUNK = ROWS_W // CHUNK
RPG = 8
GROUPS = CHUNK // RPG
WPG = RPG * LIST_LEN // L


def _build():
    mesh = plsc.VectorSubcoreMesh(core_axis_name="c", subcore_axis_name="s")

    @functools.partial(
        pl.kernel,
        mesh=mesh,
        out_type=jax.ShapeDtypeStruct((BATCH, LIST_LEN), jnp.float32),
        compiler_params=pltpu.CompilerParams(needs_layout_passes=False),
        scratch_types=[
            pltpu.VMEM((TABLE_PAD,), jnp.float32),
            pltpu.VMEM((CHUNK, LIST_LEN), jnp.int32),
            pltpu.VMEM((CHUNK, LIST_LEN), jnp.float32),
        ],
    )
    def gather_kernel(pos_hbm, model_hbm, out_hbm, table_v, idx_v, vals_v):
        wid = lax.axis_index("s") * NC + lax.axis_index("c")
        row0 = wid * ROWS_W
        pltpu.sync_copy(model_hbm, table_v.at[pl.ds(0, POSITIONS)])

        idx_g = idx_v.reshape(GROUPS, RPG, LIST_LEN)
        vals_g = vals_v.reshape(GROUPS, RPG, LIST_LEN)

        def body(g, carry):
            src = idx_g.at[g]
            dst = vals_g.at[g]
            lanes = lax.iota(jnp.int32, L)
            for k in range(WPG):
                flat = lanes + k * L
                rk = flat // LIST_LEN
                ck = flat - rk * LIST_LEN
                idx = plsc.load_gather(src, [rk, ck])
                plsc.store_scatter(dst, [rk, ck], plsc.load_gather(table_v, [idx]))
            return carry

        def chunk_body(ci, carry):
            r0 = row0 + ci * CHUNK
            pltpu.sync_copy(pos_hbm.at[pl.ds(r0, CHUNK)], idx_v)
            lax.fori_loop(0, GROUPS, body, 0)
            pltpu.sync_copy(vals_v, out_hbm.at[pl.ds(r0, CHUNK)])
            return carry

        lax.fori_loop(0, NCHUNK, chunk_body, 0)

    return gather_kernel


_GATHER = _build()


@jax.jit
def kernel(position, model):
    return _GATHER(position, model)


# speedup vs baseline: 1.5795x; 1.2251x over previous
import functools

import jax
import jax.numpy as jnp
from jax import lax
from jax.experimental import pallas as pl
from jax.experimental.pallas import tpu as pltpu
from jax.experimental.pallas import tpu_sc as plsc

BATCH = 16384
LIST_LEN = 50
POSITIONS = 50
TABLE_PAD = 64

NC, NS, L = 2, 16, 16
NW = NC * NS
ROWS_W = BATCH // NW
CHUNK = 128
NCHUNK = ROWS_W // CHUNK
NSLOT = 2
RPG = 8
GROUPS = CHUNK // RPG
WPG = RPG * LIST_LEN // L


def _build():
    mesh = plsc.VectorSubcoreMesh(core_axis_name="c", subcore_axis_name="s")

    @functools.partial(
        pl.kernel,
        mesh=mesh,
        out_type=jax.ShapeDtypeStruct((BATCH, LIST_LEN), jnp.float32),
        compiler_params=pltpu.CompilerParams(needs_layout_passes=False),
        scratch_types=[
            pltpu.VMEM((TABLE_PAD,), jnp.float32),
            pltpu.VMEM((NSLOT, CHUNK, LIST_LEN), jnp.int32),
            pltpu.VMEM((NSLOT, CHUNK, LIST_LEN), jnp.float32),
            pltpu.SemaphoreType.DMA,
            pltpu.SemaphoreType.DMA,
            pltpu.SemaphoreType.DMA,
            pltpu.SemaphoreType.DMA,
        ],
    )
    def gather_kernel(
        pos_hbm, model_hbm, out_hbm, table_v, idx_v, vals_v,
        in_sem0, in_sem1, out_sem0, out_sem1,
    ):
        wid = lax.axis_index("s") * NC + lax.axis_index("c")
        row0 = wid * ROWS_W
        in_sems = (in_sem0, in_sem1)
        out_sems = (out_sem0, out_sem1)

        pltpu.sync_copy(model_hbm, table_v.at[pl.ds(0, POSITIONS)])

        def in_copy(ci, s):
            return pltpu.make_async_copy(
                pos_hbm.at[pl.ds(row0 + ci * CHUNK, CHUNK)],
                idx_v.at[s],
                in_sems[s],
            )

        def out_copy(ci, s):
            return pltpu.make_async_copy(
                vals_v.at[s],
                out_hbm.at[pl.ds(row0 + ci * CHUNK, CHUNK)],
                out_sems[s],
            )

        in_copy(0, 0).start()
        in_copy(1, 1).start()

        for ci in range(NCHUNK):
            s = ci % NSLOT
            in_copy(ci, s).wait()
            if ci >= NSLOT:
                out_copy(ci - NSLOT, s).wait()

            src = idx_v.at[s]
            dst = vals_v.at[s]
            lanes = lax.iota(jnp.int32, L)

            @plsc.parallel_loop(0, GROUPS, unroll=2)
            def _group(g):
                sg = src.at[pl.ds(g * RPG, RPG)]
                dg = dst.at[pl.ds(g * RPG, RPG)]
                for k in range(WPG):
                    flat = lanes + k * L
                    rk = flat // LIST_LEN
                    ck = flat - rk * LIST_LEN
                    idx = plsc.load_gather(sg, [rk, ck])
                    plsc.store_scatter(
                        dg, [rk, ck], plsc.load_gather(table_v, [idx])
                    )

            if ci + NSLOT < NCHUNK:
                in_copy(ci + NSLOT, s).start()
            out_copy(ci, s).start()

        out_copy(NCHUNK - NSLOT, 0).wait()
        out_copy(NCHUNK - 1, 1).wait()

    return gather_kernel


_GATHER = _build()


@jax.jit
def kernel(position, model):
    return _GATHER(position, model)


# speedup vs baseline: 1.5857x; 1.0039x over previous
import functools

import jax
import jax.numpy as jnp
from jax import lax
from jax.experimental import pallas as pl
from jax.experimental.pallas import tpu as pltpu
from jax.experimental.pallas import tpu_sc as plsc

BATCH = 16384
LIST_LEN = 50
POSITIONS = 50
TABLE_PAD = 64

NC, NS, L = 2, 16, 16
NW = NC * NS
ROWS_W = BATCH // NW
CHUNK = 128
NCHUNK = ROWS_W // CHUNK
NSLOT = 2
RPG = 8
GROUPS = CHUNK // RPG
WPG = RPG * LIST_LEN // L


def _build():
    mesh = plsc.VectorSubcoreMesh(core_axis_name="c", subcore_axis_name="s")

    @functools.partial(
        pl.kernel,
        mesh=mesh,
        out_type=jax.ShapeDtypeStruct((BATCH, LIST_LEN), jnp.float32),
        compiler_params=pltpu.CompilerParams(needs_layout_passes=False),
        scratch_types=[
            pltpu.VMEM((TABLE_PAD,), jnp.float32),
            pltpu.VMEM((NSLOT, CHUNK, LIST_LEN), jnp.int32),
            pltpu.VMEM((NSLOT, CHUNK, LIST_LEN), jnp.float32),
            pltpu.SemaphoreType.DMA,
            pltpu.SemaphoreType.DMA,
            pltpu.SemaphoreType.DMA,
            pltpu.SemaphoreType.DMA,
        ],
    )
    def gather_kernel(
        pos_hbm, model_hbm, out_hbm, table_v, idx_v, vals_v,
        in_sem0, in_sem1, out_sem0, out_sem1,
    ):
        wid = lax.axis_index("s") * NC + lax.axis_index("c")
        row0 = wid * ROWS_W
        in_sems = (in_sem0, in_sem1)
        out_sems = (out_sem0, out_sem1)

        pltpu.sync_copy(model_hbm, table_v.at[pl.ds(0, POSITIONS)])

        def in_copy(ci, s):
            return pltpu.make_async_copy(
                pos_hbm.at[pl.ds(row0 + ci * CHUNK, CHUNK)],
                idx_v.at[s],
                in_sems[s],
            )

        def out_copy(ci, s):
            return pltpu.make_async_copy(
                vals_v.at[s],
                out_hbm.at[pl.ds(row0 + ci * CHUNK, CHUNK)],
                out_sems[s],
            )

        in_copy(0, 0).start()
        in_copy(1, 1).start()

        for ci in range(NCHUNK):
            s = ci % NSLOT
            in_copy(ci, s).wait()
            if ci >= NSLOT:
                out_copy(ci - NSLOT, s).wait()

            src = idx_v.at[s]
            dst = vals_v.at[s]
            lanes = lax.iota(jnp.int32, L)

            @plsc.parallel_loop(0, GROUPS, unroll=4)
            def _group(g):
                sg = src.at[pl.ds(g * RPG, RPG)]
                dg = dst.at[pl.ds(g * RPG, RPG)]
                for k in range(WPG):
                    flat = lanes + k * L
                    rk = flat // LIST_LEN
                    ck = flat - rk * LIST_LEN
                    idx = plsc.load_gather(sg, [rk, ck])
                    plsc.store_scatter(
                        dg, [rk, ck], plsc.load_gather(table_v, [idx])
                    )

            if ci + NSLOT < NCHUNK:
                in_copy(ci + NSLOT, s).start()
            out_copy(ci, s).start()

        out_copy(NCHUNK - NSLOT, 0).wait()
        out_copy(NCHUNK - 1, 1).wait()

    return gather_kernel


_GATHER = _build()


@jax.jit
def kernel(position, model):
    return _GATHER(position, model)


# speedup vs baseline: 2.6484x; 1.6702x over previous
import functools

import jax
import jax.numpy as jnp
from jax import lax
from jax.experimental import pallas as pl
from jax.experimental.pallas import tpu as pltpu
from jax.experimental.pallas import tpu_sc as plsc

BATCH = 16384
LIST_LEN = 50
POSITIONS = 50
TABLE_PAD = 64

NC, NS, L = 2, 16, 16
NW = NC * NS
COLS_W = BATCH // NW
CCHUNK = 256
NCHUNK = COLS_W // CCHUNK
NSLOT = 2
WPR = CCHUNK // L


def _build():
    mesh = plsc.VectorSubcoreMesh(core_axis_name="c", subcore_axis_name="s")

    @functools.partial(
        pl.kernel,
        mesh=mesh,
        out_type=jax.ShapeDtypeStruct((LIST_LEN, BATCH), jnp.float32),
        compiler_params=pltpu.CompilerParams(needs_layout_passes=False),
        scratch_types=[
            pltpu.VMEM((TABLE_PAD,), jnp.float32),
            pltpu.VMEM((NSLOT, LIST_LEN, CCHUNK), jnp.int32),
            pltpu.VMEM((NSLOT, LIST_LEN, CCHUNK), jnp.float32),
            pltpu.SemaphoreType.DMA,
            pltpu.SemaphoreType.DMA,
            pltpu.SemaphoreType.DMA,
            pltpu.SemaphoreType.DMA,
        ],
    )
    def gather_kernel(
        pos_hbm, model_hbm, out_hbm, table_v, idx_v, vals_v,
        in_sem0, in_sem1, out_sem0, out_sem1,
    ):
        wid = lax.axis_index("s") * NC + lax.axis_index("c")
        col0 = wid * COLS_W
        in_sems = (in_sem0, in_sem1)
        out_sems = (out_sem0, out_sem1)

        def in_copy(ci, s):
            return pltpu.make_async_copy(
                pos_hbm.at[:, pl.ds(col0 + ci * CCHUNK, CCHUNK)],
                idx_v.at[s],
                in_sems[s],
            )

        def out_copy(ci, s):
            return pltpu.make_async_copy(
                vals_v.at[s],
                out_hbm.at[:, pl.ds(col0 + ci * CCHUNK, CCHUNK)],
                out_sems[s],
            )

        in_copy(0, 0).start()
        in_copy(1, 1).start()
        pltpu.sync_copy(model_hbm, table_v.at[pl.ds(0, POSITIONS)])

        for ci in range(NCHUNK):
            s = ci % NSLOT
            in_copy(ci, s).wait()
            if ci >= NSLOT:
                out_copy(ci - NSLOT, s).wait()

            src = idx_v.at[s]
            dst = vals_v.at[s]

            @plsc.parallel_loop(0, LIST_LEN, unroll=2)
            def _row(r):
                for k in range(WPR):
                    idx = src[r, pl.ds(k * L, L)]
                    dst[r, pl.ds(k * L, L)] = plsc.load_gather(table_v, [idx])

            if ci + NSLOT < NCHUNK:
                in_copy(ci + NSLOT, s).start()
            out_copy(ci, s).start()

        for ci in range(max(NCHUNK - NSLOT, 0), NCHUNK):
            out_copy(ci, ci % NSLOT).wait()

    return gather_kernel


_GATHER = _build()


@jax.jit
def kernel(position, model):
    out_t = _GATHER(position.T, model)
    return out_t.T
